# baseline (speedup 1.0000x reference)
import jax
import jax.numpy as jnp
from jax import lax
from jax.experimental import pallas as pl
from jax.experimental.pallas import tpu as pltpu
from jax.experimental.pallas import tpu_sc as plsc

N = 50000
E = 800000
N_FIELDS = 4
VOCAB = 1000
HID = 64
L = 16
W = 32
NC = 2
NS = 16
NCH = HID // W

NPAD = 50688
NPT = NPAD // NS
SB = 32
NSB = NPT // SB

EPAD = 819200
EROWS = EPAD // 128
ERPT = EROWS // NS
ES = 10
NSL = ERPT // ES
EB = 2
NEB = ES // EB

_prec = lax.Precision.HIGHEST


def _sc_body(xflat, esrc, edst, wtab, out,
             acc_sp, cnt_sp,
             gb, hupd, invb, srcall, dstall, onesv, xidx, idxb,
             isem, gsem, ssem, dsem):
  cid = lax.axis_index("c")
  sid = lax.axis_index("s")
  node0 = sid * NPT
  erow0 = sid * ERPT

  zeros16 = jnp.zeros((L,), jnp.float32)
  ones16 = jnp.full((L,), 1.0, jnp.float32)

  for i in range(128 // L):
    onesv[pl.ds(i * L, L)] = ones16

  def invz_body(i, _):
    invb[pl.ds(i * L, L)] = zeros16
    return 0
  lax.fori_loop(0, NPT // L, invz_body, 0)
  pltpu.sync_copy(invb, cnt_sp.at[pl.ds(node0, NPT)])

  for q in range(SB):
    hupd[q, pl.ds(0, L)] = zeros16
    hupd[q, pl.ds(L, L)] = zeros16

  def accz_body(sb, _):
    pltpu.sync_copy(hupd, acc_sp.at[pl.ds(node0 + sb * SB, SB), :])
    return 0
  lax.fori_loop(0, NSB, accz_body, 0)
  plsc.subcore_barrier()

  def deg_body(sl, _):
    r0 = erow0 + sl * ES
    pltpu.async_copy(edst.at[pl.ds(r0, ES)], dstall, isem).wait()
    ss = [pltpu.async_copy(onesv.at[pl.ds(0, 128)],
                           cnt_sp.at[dstall.at[r]], ssem, add=True)
          for r in range(ES)]
    for s in ss:
      s.wait()
    return 0
  lax.fori_loop(0, NSL, deg_body, 0)
  plsc.subcore_barrier()

  pltpu.sync_copy(cnt_sp.at[pl.ds(node0, NPT)], invb)

  def inv_body(i, _):
    v = invb[pl.ds(i * L, L)]
    invb[pl.ds(i * L, L)] = 1.0 / jnp.maximum(v, 1.0)
    return 0
  lax.fori_loop(0, NPT // L, inv_body, 0)

  offpat = lax.rem(lax.broadcasted_iota(jnp.int32, (L,), 0),
                   jnp.int32(N_FIELDS)) * jnp.int32(VOCAB)

  def chunk_body(c, _):
    def build_body(sb, _):
      nb = node0 + sb * SB
      pltpu.async_copy(xflat.at[pl.ds(nb * N_FIELDS, SB * N_FIELDS)],
                       xidx, isem).wait()
      for q in range((SB * N_FIELDS) // L):
        idxb[pl.ds(q * L, L)] = xidx[pl.ds(q * L, L)] + offpat
      gs = [pltpu.async_copy(wtab.at[cid, c].at[idxb.at[pl.ds(g * 128, 128)]],
                             gb.at[pl.ds(g * 128, 128), :], gsem)
            for g in range((SB * N_FIELDS) // 128)]
      for g in gs:
        g.wait()
      for n in range(SB):
        for hh in range(W // L):
          s = hh * L
          a = gb[n * N_FIELDS, pl.ds(s, L)] + gb[n * N_FIELDS + 1,
                                                 pl.ds(s, L)]
          b = gb[n * N_FIELDS + 2, pl.ds(s, L)] + gb[n * N_FIELDS + 3,
                                                     pl.ds(s, L)]
          hupd[n, pl.ds(s, L)] = a + b
      pltpu.async_copy(hupd, out.at[cid, c, pl.ds(nb, SB)], dsem).wait()
      return 0
    lax.fori_loop(0, NSB, build_body, 0)
    plsc.subcore_barrier()

    for layer in range(2):
      def agg_body(sl, _):
        r0 = erow0 + sl * ES
        ci = pltpu.async_copy(esrc.at[pl.ds(r0, ES)], srcall, isem)
        cj = pltpu.async_copy(edst.at[pl.ds(r0, ES)], dstall, dsem)
        ci.wait()
        cj.wait()
        pend = [None, None]
        for b in range(NEB):
          p = b % 2
          if pend[p] is not None:
            for s in pend[p]:
              s.wait()
          gs = [pltpu.async_copy(out.at[cid, c].at[srcall.at[b * EB + j]],
                                 gb.at[pl.ds((p * EB + j) * 128, 128), :],
                                 gsem)
                for j in range(EB)]
          for g in gs:
            g.wait()
          pend[p] = [
              pltpu.async_copy(gb.at[pl.ds((p * EB + j) * 128, 128), :],
                               acc_sp.at[dstall.at[b * EB + j]], ssem,
                               add=True)
              for j in range(EB)]
        for p in range(2):
          for s in pend[p]:
            s.wait()
        return 0
      lax.fori_loop(0, NSL, agg_body, 0)
      plsc.subcore_barrier()

      def upd_body(sb, _):
        nb = node0 + sb * SB
        ch = pltpu.async_copy(out.at[cid, c, pl.ds(nb, SB)], hupd, isem)
        ca = pltpu.async_copy(acc_sp.at[pl.ds(nb, SB), :],
                              gb.at[pl.ds(0, SB), :], gsem)
        ch.wait()
        ca.wait()
        for q in range(SB // L):
          ivv = invb[pl.ds(sb * SB + q * L, L)]
          for k in range(L):
            n = q * L + k
            for hh in range(W // L):
              s = hh * L
              r = hupd[n, pl.ds(s, L)] + gb[n, pl.ds(s, L)] * ivv[k]
              if layer == 0:
                r = jnp.maximum(r, 0.0)
              hupd[n, pl.ds(s, L)] = r
              gb[n, pl.ds(s, L)] = zeros16
        co = pltpu.async_copy(hupd, out.at[cid, c, pl.ds(nb, SB)], dsem)
        cz = pltpu.async_copy(gb.at[pl.ds(0, SB), :],
                              acc_sp.at[pl.ds(nb, SB), :], ssem)
        co.wait()
        cz.wait()
        return 0
      lax.fori_loop(0, NSB, upd_body, 0)
      plsc.subcore_barrier()
    return 0
  lax.fori_loop(0, NCH, chunk_body, 0)


_sc_agg = pl.kernel(
    _sc_body,
    out_type=jax.ShapeDtypeStruct((2, NCH, NPAD, W), jnp.float32),
    mesh=plsc.VectorSubcoreMesh(core_axis_name="c", subcore_axis_name="s",
                                num_cores=NC, num_subcores=NS),
    compiler_params=pltpu.CompilerParams(use_tc_tiling_on_sc=False),
    scratch_types=[
        pltpu.VMEM_SHARED((NPAD, W), jnp.float32),
        pltpu.VMEM_SHARED((NPAD,), jnp.float32),
        pltpu.VMEM((2 * EB * 128, W), jnp.float32),
        pltpu.VMEM((SB, W), jnp.float32),
        pltpu.VMEM((NPT,), jnp.float32),
        pltpu.VMEM((ES, 128), jnp.int32),
        pltpu.VMEM((ES, 128), jnp.int32),
        pltpu.VMEM((128,), jnp.float32),
        pltpu.VMEM((SB * N_FIELDS,), jnp.int32),
        pltpu.VMEM((SB * N_FIELDS,), jnp.int32),
        pltpu.SemaphoreType.DMA,
        pltpu.SemaphoreType.DMA,
        pltpu.SemaphoreType.DMA,
        pltpu.SemaphoreType.DMA,
    ],
)


def _post_body(hs_ref, hc_ref, mss_ref, mcs_ref, msc_ref, mcc_ref,
               bs_ref, bc_ref, zs_ref, zc_ref):
  hs = hs_ref[...]
  hc = hc_ref[...]
  zs_ref[...] = (jnp.dot(hs, mss_ref[...], precision=_prec)
                 + jnp.dot(hc, mcs_ref[...], precision=_prec)
                 + bs_ref[...])
  zc_ref[...] = (jnp.dot(hs, msc_ref[...], precision=_prec)
                 + jnp.dot(hc, mcc_ref[...], precision=_prec)
                 + bc_ref[...])


_BN = 512


def _post_call(hs, hc, mss, mcs, msc, mcc, bs, bc):
  grid = (NPAD // _BN,)
  row_spec = pl.BlockSpec((_BN, HID), lambda i: (i, 0))
  w_spec = pl.BlockSpec((HID, HID), lambda i: (0, 0))
  b_spec = pl.BlockSpec((1, HID), lambda i: (0, 0))
  return pl.pallas_call(
      _post_body,
      grid=grid,
      in_specs=[row_spec, row_spec, w_spec, w_spec, w_spec, w_spec,
                b_spec, b_spec],
      out_specs=[row_spec, row_spec],
      out_shape=[jax.ShapeDtypeStruct((NPAD, HID), jnp.float32),
                 jax.ShapeDtypeStruct((NPAD, HID), jnp.float32)],
  )(hs, hc, mss, mcs, msc, mcc, bs, bc)


def kernel(x, edge_index, emb_sim, emb_cor, W_in_sim, b_in_sim, W_out_sim,
           b_out_sim, W_in_cor, b_in_cor, W_out_cor, b_out_cor, W_s2c,
           W_c2s, a1, a2, b2):
  def build_tab(emb, W_in, b_in):
    full = jnp.einsum("fve,feh->fvh", emb,
                      W_in.reshape(N_FIELDS, L, HID),
                      precision=_prec)
    full = full + b_in[None, None, :] / N_FIELDS
    full = full.reshape(N_FIELDS * VOCAB, NCH, W)
    return full.transpose(1, 0, 2)

  wtab = jnp.stack([build_tab(emb_sim, W_in_sim, b_in_sim),
                    build_tab(emb_cor, W_in_cor, b_in_cor)])

  xflat = jnp.pad(x, ((0, NPAD - N), (0, 0))).reshape(-1)
  pad_ids = (N + (jnp.arange(EPAD - E, dtype=jnp.int32) % (NPAD - N)))
  esrc = jnp.concatenate([edge_index[0], pad_ids]).reshape(EROWS, 128)
  edst = jnp.concatenate([edge_index[1], pad_ids]).reshape(EROWS, 128)

  h2 = _sc_agg(xflat, esrc, edst, wtab)
  h2s = h2[0].transpose(1, 0, 2).reshape(NPAD, HID)
  h2c = h2[1].transpose(1, 0, 2).reshape(NPAD, HID)

  s1, s2, s3 = a1[0], a2[0], b2[0]
  eye = jnp.eye(HID, dtype=jnp.float32)
  a_ss = (1.0 - s2 - s3) * eye + (s1 * s3) * jnp.dot(W_s2c, W_c2s,
                                                     precision=_prec)
  a_cc = (1.0 - s2 - s3) * eye + (s1 * s3) * jnp.dot(W_c2s, W_s2c,
                                                     precision=_prec)
  a_cs = (s2 + s3 * (1.0 - s1)) * W_c2s
  a_sc = (s2 + s3 * (1.0 - s1)) * W_s2c
  mss = jnp.dot(W_out_sim, a_ss, precision=_prec)
  mcs = jnp.dot(W_out_cor, a_cs, precision=_prec)
  msc = jnp.dot(W_out_sim, a_sc, precision=_prec)
  mcc = jnp.dot(W_out_cor, a_cc, precision=_prec)
  bs = (jnp.dot(b_out_sim, a_ss, precision=_prec)
        + jnp.dot(b_out_cor, a_cs, precision=_prec)).reshape(1, HID)
  bc = (jnp.dot(b_out_sim, a_sc, precision=_prec)
        + jnp.dot(b_out_cor, a_cc, precision=_prec)).reshape(1, HID)

  zs, zc = _post_call(h2s, h2c, mss, mcs, msc, mcc, bs, bc)
  return zs[:N], zc[:N]


# speedup vs baseline: 1.0018x; 1.0018x over previous
import jax
import jax.numpy as jnp
from jax import lax
from jax.experimental import pallas as pl
from jax.experimental.pallas import tpu as pltpu
from jax.experimental.pallas import tpu_sc as plsc

N = 50000
E = 800000
N_FIELDS = 4
VOCAB = 1000
HID = 64
L = 16
NC = 2
NS = 16
NCH = HID // L

NPAD = 51200
NPT = NPAD // NS
SB = 320
NSB = NPT // SB

EPAD = 819200
EROWS = EPAD // 128
ERPT = EROWS // NS
ES = 50
NSL = ERPT // ES
EB = 5
NEB = ES // EB

_prec = lax.Precision.HIGHEST


def _sc_body(xflat, esrc, edst, wtab, out,
             acc_sp, cnt_sp,
             gb, hupd, aupd, invb, srcall, dstall, onesv, xidx, idxb,
             isem, gsem, ssem, dsem):
  cid = lax.axis_index("c")
  sid = lax.axis_index("s")
  node0 = sid * NPT
  erow0 = sid * ERPT

  zeros16 = jnp.zeros((L,), jnp.float32)
  ones16 = jnp.full((L,), 1.0, jnp.float32)

  for i in range(128 // L):
    onesv[pl.ds(i * L, L)] = ones16

  def invz_body(i, _):
    invb[pl.ds(i * L, L)] = zeros16
    return 0
  lax.fori_loop(0, NPT // L, invz_body, 0)
  pltpu.sync_copy(invb, cnt_sp.at[pl.ds(node0, NPT)])

  for q in range(SB):
    aupd[q, :] = zeros16

  def accz_body(sb, _):
    pltpu.sync_copy(aupd, acc_sp.at[pl.ds(node0 + sb * SB, SB), :])
    return 0
  lax.fori_loop(0, NSB, accz_body, 0)
  plsc.subcore_barrier()

  def deg_body(sl, _):
    r0 = erow0 + sl * ES
    pltpu.async_copy(edst.at[pl.ds(r0, ES)], dstall, isem).wait()
    ss = [pltpu.async_copy(onesv.at[pl.ds(0, 128)],
                           cnt_sp.at[dstall.at[r]], ssem, add=True)
          for r in range(ES)]
    for s in ss:
      s.wait()
    return 0
  lax.fori_loop(0, NSL, deg_body, 0)
  plsc.subcore_barrier()

  pltpu.sync_copy(cnt_sp.at[pl.ds(node0, NPT)], invb)

  def inv_body(i, _):
    v = invb[pl.ds(i * L, L)]
    invb[pl.ds(i * L, L)] = 1.0 / jnp.maximum(v, 1.0)
    return 0
  lax.fori_loop(0, NPT // L, inv_body, 0)

  offpat = lax.rem(lax.broadcasted_iota(jnp.int32, (L,), 0),
                   jnp.int32(N_FIELDS)) * jnp.int32(VOCAB)

  def chunk_body(c, _):
    def build_body(sb, _):
      nb = node0 + sb * SB
      pltpu.async_copy(xflat.at[pl.ds(nb * N_FIELDS, SB * N_FIELDS)],
                       xidx, isem).wait()
      for q in range((SB * N_FIELDS) // L):
        idxb[pl.ds(q * L, L)] = xidx[pl.ds(q * L, L)] + offpat
      gs = [pltpu.async_copy(wtab.at[cid, c].at[idxb.at[pl.ds(g * 128, 128)]],
                             gb.at[pl.ds(g * 128, 128), :], gsem)
            for g in range((SB * N_FIELDS) // 128)]
      for g in gs:
        g.wait()
      for n in range(SB):
        a = gb[n * N_FIELDS, :] + gb[n * N_FIELDS + 1, :]
        b = gb[n * N_FIELDS + 2, :] + gb[n * N_FIELDS + 3, :]
        hupd[n, :] = a + b
      pltpu.async_copy(hupd, out.at[cid, c, pl.ds(nb, SB)], dsem).wait()
      return 0
    lax.fori_loop(0, NSB, build_body, 0)
    plsc.subcore_barrier()

    for layer in range(2):
      def agg_body(sl, _):
        r0 = erow0 + sl * ES
        ci = pltpu.async_copy(esrc.at[pl.ds(r0, ES)], srcall, isem)
        cj = pltpu.async_copy(edst.at[pl.ds(r0, ES)], dstall, dsem)
        ci.wait()
        cj.wait()
        pend = [None, None]
        for b in range(NEB):
          p = b % 2
          if pend[p] is not None:
            for s in pend[p]:
              s.wait()
          gs = [pltpu.async_copy(out.at[cid, c].at[srcall.at[b * EB + j]],
                                 gb.at[pl.ds((p * EB + j) * 128, 128), :],
                                 gsem)
                for j in range(EB)]
          for g in gs:
            g.wait()
          pend[p] = [
              pltpu.async_copy(gb.at[pl.ds((p * EB + j) * 128, 128), :],
                               acc_sp.at[dstall.at[b * EB + j]], ssem,
                               add=True)
              for j in range(EB)]
        for p in range(2):
          for s in pend[p]:
            s.wait()
        return 0
      lax.fori_loop(0, NSL, agg_body, 0)
      plsc.subcore_barrier()

      def upd_body(sb, _):
        nb = node0 + sb * SB
        ch = pltpu.async_copy(out.at[cid, c, pl.ds(nb, SB)], hupd, isem)
        ca = pltpu.async_copy(acc_sp.at[pl.ds(nb, SB), :], aupd, gsem)
        ch.wait()
        ca.wait()
        for q in range(SB // L):
          ivv = invb[pl.ds(sb * SB + q * L, L)]
          for k in range(L):
            n = q * L + k
            r = hupd[n, :] + aupd[n, :] * ivv[k]
            if layer == 0:
              r = jnp.maximum(r, 0.0)
            hupd[n, :] = r
            aupd[n, :] = zeros16
        co = pltpu.async_copy(hupd, out.at[cid, c, pl.ds(nb, SB)], dsem)
        cz = pltpu.async_copy(aupd, acc_sp.at[pl.ds(nb, SB), :], ssem)
        co.wait()
        cz.wait()
        return 0
      lax.fori_loop(0, NSB, upd_body, 0)
      plsc.subcore_barrier()
    return 0
  lax.fori_loop(0, NCH, chunk_body, 0)


_sc_agg = pl.kernel(
    _sc_body,
    out_type=jax.ShapeDtypeStruct((2, NCH, NPAD, L), jnp.float32),
    mesh=plsc.VectorSubcoreMesh(core_axis_name="c", subcore_axis_name="s",
                                num_cores=NC, num_subcores=NS),
    compiler_params=pltpu.CompilerParams(use_tc_tiling_on_sc=False),
    scratch_types=[
        pltpu.VMEM_SHARED((NPAD, L), jnp.float32),
        pltpu.VMEM_SHARED((NPAD,), jnp.float32),
        pltpu.VMEM((SB * N_FIELDS, L), jnp.float32),
        pltpu.VMEM((SB, L), jnp.float32),
        pltpu.VMEM((SB, L), jnp.float32),
        pltpu.VMEM((NPT,), jnp.float32),
        pltpu.VMEM((ES, 128), jnp.int32),
        pltpu.VMEM((ES, 128), jnp.int32),
        pltpu.VMEM((128,), jnp.float32),
        pltpu.VMEM((SB * N_FIELDS,), jnp.int32),
        pltpu.VMEM((SB * N_FIELDS,), jnp.int32),
        pltpu.SemaphoreType.DMA,
        pltpu.SemaphoreType.DMA,
        pltpu.SemaphoreType.DMA,
        pltpu.SemaphoreType.DMA,
    ],
)


def _post_body(hs_ref, hc_ref, mss_ref, mcs_ref, msc_ref, mcc_ref,
               bs_ref, bc_ref, zs_ref, zc_ref):
  hs = hs_ref[...]
  hc = hc_ref[...]
  zs_ref[...] = (jnp.dot(hs, mss_ref[...], precision=_prec)
                 + jnp.dot(hc, mcs_ref[...], precision=_prec)
                 + bs_ref[...])
  zc_ref[...] = (jnp.dot(hs, msc_ref[...], precision=_prec)
                 + jnp.dot(hc, mcc_ref[...], precision=_prec)
                 + bc_ref[...])


_BN = 512


def _post_call(hs, hc, mss, mcs, msc, mcc, bs, bc):
  grid = (NPAD // _BN,)
  row_spec = pl.BlockSpec((_BN, HID), lambda i: (i, 0))
  w_spec = pl.BlockSpec((HID, HID), lambda i: (0, 0))
  b_spec = pl.BlockSpec((1, HID), lambda i: (0, 0))
  return pl.pallas_call(
      _post_body,
      grid=grid,
      in_specs=[row_spec, row_spec, w_spec, w_spec, w_spec, w_spec,
                b_spec, b_spec],
      out_specs=[row_spec, row_spec],
      out_shape=[jax.ShapeDtypeStruct((NPAD, HID), jnp.float32),
                 jax.ShapeDtypeStruct((NPAD, HID), jnp.float32)],
  )(hs, hc, mss, mcs, msc, mcc, bs, bc)


def kernel(x, edge_index, emb_sim, emb_cor, W_in_sim, b_in_sim, W_out_sim,
           b_out_sim, W_in_cor, b_in_cor, W_out_cor, b_out_cor, W_s2c,
           W_c2s, a1, a2, b2):
  def build_tab(emb, W_in, b_in):
    full = jnp.einsum("fve,feh->fvh", emb,
                      W_in.reshape(N_FIELDS, L, HID),
                      precision=_prec)
    full = full + b_in[None, None, :] / N_FIELDS
    full = full.reshape(N_FIELDS * VOCAB, NCH, L)
    return full.transpose(1, 0, 2)

  wtab = jnp.stack([build_tab(emb_sim, W_in_sim, b_in_sim),
                    build_tab(emb_cor, W_in_cor, b_in_cor)])

  xflat = jnp.pad(x, ((0, NPAD - N), (0, 0))).reshape(-1)
  pad_ids = (N + (jnp.arange(EPAD - E, dtype=jnp.int32) % (NPAD - N)))
  esrc = jnp.concatenate([edge_index[0], pad_ids]).reshape(EROWS, 128)
  edst = jnp.concatenate([edge_index[1], pad_ids]).reshape(EROWS, 128)

  h2 = _sc_agg(xflat, esrc, edst, wtab)
  h2s = h2[0].transpose(1, 0, 2).reshape(NPAD, HID)
  h2c = h2[1].transpose(1, 0, 2).reshape(NPAD, HID)

  s1, s2, s3 = a1[0], a2[0], b2[0]
  eye = jnp.eye(HID, dtype=jnp.float32)
  a_ss = (1.0 - s2 - s3) * eye + (s1 * s3) * jnp.dot(W_s2c, W_c2s,
                                                     precision=_prec)
  a_cc = (1.0 - s2 - s3) * eye + (s1 * s3) * jnp.dot(W_c2s, W_s2c,
                                                     precision=_prec)
  a_cs = (s2 + s3 * (1.0 - s1)) * W_c2s
  a_sc = (s2 + s3 * (1.0 - s1)) * W_s2c
  mss = jnp.dot(W_out_sim, a_ss, precision=_prec)
  mcs = jnp.dot(W_out_cor, a_cs, precision=_prec)
  msc = jnp.dot(W_out_sim, a_sc, precision=_prec)
  mcc = jnp.dot(W_out_cor, a_cc, precision=_prec)
  bs = (jnp.dot(b_out_sim, a_ss, precision=_prec)
        + jnp.dot(b_out_cor, a_cs, precision=_prec)).reshape(1, HID)
  bc = (jnp.dot(b_out_sim, a_sc, precision=_prec)
        + jnp.dot(b_out_cor, a_cc, precision=_prec)).reshape(1, HID)

  zs, zc = _post_call(h2s, h2c, mss, mcs, msc, mcc, bs, bc)
  return zs[:N], zc[:N]
